# baseline (speedup 1.0000x reference)
import functools

import jax
import jax.numpy as jnp
import numpy as np
from jax import lax
from jax.experimental import pallas as pl
from jax.experimental.pallas import tpu as pltpu
from jax.experimental.pallas import tpu_sc as plsc

TOTAL_TOKENS = 1024
RETAIN = 256
BATCH = 64
C = 768

ROWS = BATCH * TOTAL_TOKENS
ROWS_Z = BATCH * (TOTAL_TOKENS - RETAIN)
NW = 32
RPW = ROWS_Z // NW
CHUNK = 32
NCH = RPW // CHUNK
NBUF = 4

ZT = TOTAL_TOKENS - RETAIN
W = 4
L = 3


def _build_z_kernel():
    info = plsc.get_sparse_core_info()
    nc = info.num_cores
    mesh = plsc.VectorSubcoreMesh(core_axis_name="c", subcore_axis_name="s")

    @functools.partial(
        pl.kernel,
        mesh=mesh,
        out_type=jax.ShapeDtypeStruct((ROWS_Z, C), jnp.float32),
        scratch_types=(
            [pltpu.VMEM((NCH, CHUNK), jnp.int32)]
            + [pltpu.VMEM((CHUNK, C), jnp.float32) for _ in range(NBUF)]
            + [pltpu.SemaphoreType.DMA for _ in range(2 * NBUF)]
        ),
    )
    def z_kernel(x_hbm, idx_hbm, z_hbm, idx_v, *bufs_and_sems):
        bufs = bufs_and_sems[:NBUF]
        gsem = bufs_and_sems[NBUF : 2 * NBUF]
        ssem = bufs_and_sems[2 * NBUF :]
        w = lax.axis_index("s") * nc + lax.axis_index("c")
        pltpu.sync_copy(idx_hbm.at[w], idx_v)
        obase = w * RPW

        def gather(c, b):
            return pltpu.make_async_copy(x_hbm.at[idx_v.at[c]], bufs[b], gsem[b])

        def store(c, b):
            return pltpu.make_async_copy(
                bufs[b], z_hbm.at[pl.ds(obase + c * CHUNK, CHUNK)], ssem[b]
            )

        for b in range(NBUF - 1):
            gather(b, b).start()

        def body(i, carry):
            for b in range(NBUF):
                cc = NBUF * i + b
                gather(cc, b).wait()
                store(cc, b).start()
                nb = (b + NBUF - 1) % NBUF

                @pl.when(cc + NBUF - 1 < NCH)
                def _():
                    @pl.when(cc >= 1)
                    def _():
                        store(cc - 1, nb).wait()

                    gather(cc + NBUF - 1, nb).start()

            return carry

        lax.fori_loop(0, NCH // NBUF, body, 0)
        for b in range(NBUF):
            store(NCH - NBUF + b, (NCH - NBUF + b) % NBUF).wait()

    return z_kernel


_z_kernel = _build_z_kernel()


def _y_tc_kernel(idx_ref, x_ref, y_ref, *bufs_and_sems):
    bufs = bufs_and_sems[:W]
    isem = bufs_and_sems[W : 2 * W]
    osem = bufs_and_sems[2 * W :]

    def copy_in(t, k):
        return pltpu.make_async_copy(x_ref.at[:, idx_ref[t]], bufs[k], isem[k])

    def copy_out(t, k):
        return pltpu.make_async_copy(bufs[k], y_ref.at[:, t], osem[k])

    for t in range(L):
        copy_in(t, t % W).start()

    def body(i, carry):
        for b in range(W):
            t = W * i + b
            copy_in(t, b).wait()
            copy_out(t, b).start()
            tt = t + L
            kk = (b + L) % W

            @pl.when(tt < RETAIN)
            def _():
                @pl.when(tt >= W)
                def _():
                    copy_out(tt - W, kk).wait()

                copy_in(tt, kk).start()

        return carry

    lax.fori_loop(0, RETAIN // W, body, 0)
    for b in range(W):
        copy_out(RETAIN - W + b, (RETAIN - W + b) % W).wait()


_y_copy = pl.pallas_call(
    _y_tc_kernel,
    out_shape=jax.ShapeDtypeStruct((BATCH, RETAIN, C), jnp.float32),
    in_specs=[
        pl.BlockSpec(memory_space=pltpu.SMEM),
        pl.BlockSpec(memory_space=pl.ANY),
    ],
    out_specs=pl.BlockSpec(memory_space=pl.ANY),
    scratch_shapes=(
        [pltpu.VMEM((BATCH, C), jnp.float32) for _ in range(W)]
        + [pltpu.SemaphoreType.DMA for _ in range(2 * W)]
    ),
)


_PERM = np.asarray(jax.random.permutation(jax.random.key(1), TOTAL_TOKENS))
_ROW_BASE = (np.arange(BATCH, dtype=np.int64) * TOTAL_TOKENS)[:, None]
_IDX_Z = (
    (_ROW_BASE + _PERM[None, RETAIN:])
    .reshape(-1)
    .astype(np.int32)
    .reshape(NW, NCH, CHUNK)
)
_IDX_Y = _PERM[:RETAIN].astype(np.int32)


def kernel(x):
    z_flat = _z_kernel(x.reshape(ROWS, C), jnp.asarray(_IDX_Z))
    y = _y_copy(jnp.asarray(_IDX_Y), x)
    return (y, z_flat.reshape(BATCH, ZT, C))


# speedup vs baseline: 1.2552x; 1.2552x over previous
import functools

import jax
import jax.numpy as jnp
import numpy as np
from jax import lax
from jax.experimental import pallas as pl
from jax.experimental.pallas import tpu as pltpu
from jax.experimental.pallas import tpu_sc as plsc

TOTAL_TOKENS = 1024
RETAIN = 256
BATCH = 64
C = 768

ROWS = BATCH * TOTAL_TOKENS
ROWS_Z = BATCH * (TOTAL_TOKENS - RETAIN)
NW = 32
RPW = ROWS_Z // NW
CHUNK = 32
NCH = RPW // CHUNK
NBUF = 4

ZT = TOTAL_TOKENS - RETAIN
W = 32
L = 24


def _build_z_kernel():
    info = plsc.get_sparse_core_info()
    nc = info.num_cores
    mesh = plsc.VectorSubcoreMesh(core_axis_name="c", subcore_axis_name="s")

    @functools.partial(
        pl.kernel,
        mesh=mesh,
        out_type=jax.ShapeDtypeStruct((ROWS_Z, C), jnp.float32),
        scratch_types=(
            [pltpu.VMEM((NCH, CHUNK), jnp.int32)]
            + [pltpu.VMEM((CHUNK, C), jnp.float32) for _ in range(NBUF)]
            + [pltpu.SemaphoreType.DMA for _ in range(2 * NBUF)]
        ),
    )
    def z_kernel(x_hbm, idx_hbm, z_hbm, idx_v, *bufs_and_sems):
        bufs = bufs_and_sems[:NBUF]
        gsem = bufs_and_sems[NBUF : 2 * NBUF]
        ssem = bufs_and_sems[2 * NBUF :]
        w = lax.axis_index("s") * nc + lax.axis_index("c")
        pltpu.sync_copy(idx_hbm.at[w], idx_v)
        obase = w * RPW

        def gather(c, b):
            return pltpu.make_async_copy(x_hbm.at[idx_v.at[c]], bufs[b], gsem[b])

        def store(c, b):
            return pltpu.make_async_copy(
                bufs[b], z_hbm.at[pl.ds(obase + c * CHUNK, CHUNK)], ssem[b]
            )

        for b in range(NBUF - 1):
            gather(b, b).start()

        def body(i, carry):
            for b in range(NBUF):
                cc = NBUF * i + b
                gather(cc, b).wait()
                store(cc, b).start()
                nb = (b + NBUF - 1) % NBUF

                @pl.when(cc + NBUF - 1 < NCH)
                def _():
                    @pl.when(cc >= 1)
                    def _():
                        store(cc - 1, nb).wait()

                    gather(cc + NBUF - 1, nb).start()

            return carry

        lax.fori_loop(0, NCH // NBUF, body, 0)
        for b in range(NBUF):
            store(NCH - NBUF + b, (NCH - NBUF + b) % NBUF).wait()

    return z_kernel


_z_kernel = _build_z_kernel()


def _y_tc_kernel(idx_ref, x_ref, y_ref, *bufs_and_sems):
    bufs = bufs_and_sems[:W]
    isem = bufs_and_sems[W : 2 * W]
    osem = bufs_and_sems[2 * W :]

    def copy_in(t, k):
        return pltpu.make_async_copy(x_ref.at[:, idx_ref[t]], bufs[k], isem[k])

    def copy_out(t, k):
        return pltpu.make_async_copy(bufs[k], y_ref.at[:, t], osem[k])

    for t in range(L):
        copy_in(t, t % W).start()

    def body(i, carry):
        for b in range(W):
            t = W * i + b
            copy_in(t, b).wait()
            copy_out(t, b).start()
            tt = t + L
            kk = (b + L) % W

            @pl.when(tt < RETAIN)
            def _():
                @pl.when(tt >= W)
                def _():
                    copy_out(tt - W, kk).wait()

                copy_in(tt, kk).start()

        return carry

    lax.fori_loop(0, RETAIN // W, body, 0)
    for b in range(W):
        copy_out(RETAIN - W + b, (RETAIN - W + b) % W).wait()


_y_copy = pl.pallas_call(
    _y_tc_kernel,
    out_shape=jax.ShapeDtypeStruct((BATCH, RETAIN, C), jnp.float32),
    in_specs=[
        pl.BlockSpec(memory_space=pltpu.SMEM),
        pl.BlockSpec(memory_space=pl.ANY),
    ],
    out_specs=pl.BlockSpec(memory_space=pl.ANY),
    scratch_shapes=(
        [pltpu.VMEM((BATCH, C), jnp.float32) for _ in range(W)]
        + [pltpu.SemaphoreType.DMA for _ in range(2 * W)]
    ),
)


_PERM = np.asarray(jax.random.permutation(jax.random.key(1), TOTAL_TOKENS))
_ROW_BASE = (np.arange(BATCH, dtype=np.int64) * TOTAL_TOKENS)[:, None]
_IDX_Z = (
    (_ROW_BASE + _PERM[None, RETAIN:])
    .reshape(-1)
    .astype(np.int32)
    .reshape(NW, NCH, CHUNK)
)
_IDX_Y = _PERM[:RETAIN].astype(np.int32)


def kernel(x):
    z_flat = _z_kernel(x.reshape(ROWS, C), jnp.asarray(_IDX_Z))
    y = _y_copy(jnp.asarray(_IDX_Y), x)
    return (y, z_flat.reshape(BATCH, ZT, C))


# speedup vs baseline: 1.2558x; 1.0005x over previous
import functools

import jax
import jax.numpy as jnp
import numpy as np
from jax import lax
from jax.experimental import pallas as pl
from jax.experimental.pallas import tpu as pltpu
from jax.experimental.pallas import tpu_sc as plsc

TOTAL_TOKENS = 1024
RETAIN = 256
BATCH = 64
C = 768

ROWS = BATCH * TOTAL_TOKENS
ROWS_Z = BATCH * (TOTAL_TOKENS - RETAIN)
NW = 32
RPW = ROWS_Z // NW
CHUNK = 32
NCH = RPW // CHUNK
NBUF = 4

ZT = TOTAL_TOKENS - RETAIN
W = 64
L = 48


def _build_z_kernel():
    info = plsc.get_sparse_core_info()
    nc = info.num_cores
    mesh = plsc.VectorSubcoreMesh(core_axis_name="c", subcore_axis_name="s")

    @functools.partial(
        pl.kernel,
        mesh=mesh,
        out_type=jax.ShapeDtypeStruct((ROWS_Z, C), jnp.float32),
        scratch_types=(
            [pltpu.VMEM((NCH, CHUNK), jnp.int32)]
            + [pltpu.VMEM((CHUNK, C), jnp.float32) for _ in range(NBUF)]
            + [pltpu.SemaphoreType.DMA for _ in range(2 * NBUF)]
        ),
    )
    def z_kernel(x_hbm, idx_hbm, z_hbm, idx_v, *bufs_and_sems):
        bufs = bufs_and_sems[:NBUF]
        gsem = bufs_and_sems[NBUF : 2 * NBUF]
        ssem = bufs_and_sems[2 * NBUF :]
        w = lax.axis_index("s") * nc + lax.axis_index("c")
        pltpu.sync_copy(idx_hbm.at[w], idx_v)
        obase = w * RPW

        def gather(c, b):
            return pltpu.make_async_copy(x_hbm.at[idx_v.at[c]], bufs[b], gsem[b])

        def store(c, b):
            return pltpu.make_async_copy(
                bufs[b], z_hbm.at[pl.ds(obase + c * CHUNK, CHUNK)], ssem[b]
            )

        for b in range(NBUF - 1):
            gather(b, b).start()

        def body(i, carry):
            for b in range(NBUF):
                cc = NBUF * i + b
                gather(cc, b).wait()
                store(cc, b).start()
                nb = (b + NBUF - 1) % NBUF

                @pl.when(cc + NBUF - 1 < NCH)
                def _():
                    @pl.when(cc >= 1)
                    def _():
                        store(cc - 1, nb).wait()

                    gather(cc + NBUF - 1, nb).start()

            return carry

        lax.fori_loop(0, NCH // NBUF, body, 0)
        for b in range(NBUF):
            store(NCH - NBUF + b, (NCH - NBUF + b) % NBUF).wait()

    return z_kernel


_z_kernel = _build_z_kernel()


def _y_tc_kernel(idx_ref, x_ref, y_ref, *bufs_and_sems):
    bufs = bufs_and_sems[:W]
    isem = bufs_and_sems[W : 2 * W]
    osem = bufs_and_sems[2 * W :]

    def copy_in(t, k):
        return pltpu.make_async_copy(x_ref.at[:, idx_ref[t]], bufs[k], isem[k])

    def copy_out(t, k):
        return pltpu.make_async_copy(bufs[k], y_ref.at[:, t], osem[k])

    for t in range(L):
        copy_in(t, t % W).start()

    def body(i, carry):
        for b in range(W):
            t = W * i + b
            copy_in(t, b).wait()
            copy_out(t, b).start()
            tt = t + L
            kk = (b + L) % W

            @pl.when(tt < RETAIN)
            def _():
                @pl.when(tt >= W)
                def _():
                    copy_out(tt - W, kk).wait()

                copy_in(tt, kk).start()

        return carry

    lax.fori_loop(0, RETAIN // W, body, 0)
    for b in range(W):
        copy_out(RETAIN - W + b, (RETAIN - W + b) % W).wait()


_y_copy = pl.pallas_call(
    _y_tc_kernel,
    out_shape=jax.ShapeDtypeStruct((BATCH, RETAIN, C), jnp.float32),
    in_specs=[
        pl.BlockSpec(memory_space=pltpu.SMEM),
        pl.BlockSpec(memory_space=pl.ANY),
    ],
    out_specs=pl.BlockSpec(memory_space=pl.ANY),
    scratch_shapes=(
        [pltpu.VMEM((BATCH, C), jnp.float32) for _ in range(W)]
        + [pltpu.SemaphoreType.DMA for _ in range(2 * W)]
    ),
)


_PERM = np.asarray(jax.random.permutation(jax.random.key(1), TOTAL_TOKENS))
_ROW_BASE = (np.arange(BATCH, dtype=np.int64) * TOTAL_TOKENS)[:, None]
_IDX_Z = (
    (_ROW_BASE + _PERM[None, RETAIN:])
    .reshape(-1)
    .astype(np.int32)
    .reshape(NW, NCH, CHUNK)
)
_IDX_Y = _PERM[:RETAIN].astype(np.int32)


def kernel(x):
    z_flat = _z_kernel(x.reshape(ROWS, C), jnp.asarray(_IDX_Z))
    y = _y_copy(jnp.asarray(_IDX_Y), x)
    return (y, z_flat.reshape(BATCH, ZT, C))
